# Initial kernel scaffold; baseline (speedup 1.0000x reference)
import jax
import jax.numpy as jnp
from jax.experimental import pallas as pl


def kernel(x, edge_index, W1, b1, W2, b2):
    raise NotImplementedError("write your pallas kernel here")


# speedup vs baseline: 9.0176x; 9.0176x over previous
import functools

import jax
import jax.numpy as jnp
from jax import lax
from jax.experimental import pallas as pl
from jax.experimental.pallas import tpu as pltpu
from jax.experimental.pallas import tpu_sc as plsc

N_NODES = 10000
IN_FEATS = 128
N_HIDDEN = 128
N_CLASSES = 64
N_EDGES = 320000

NC = 2
NS = 16
NW = NC * NS
LANES = 16

B = 128
NCHUNK = -(-N_EDGES // (NW * B))
E_PAD = NW * B * NCHUNK
RPT = 640
R = NS * RPT

_f32 = jnp.float32


def _agg_body(nfeat, count_deg, x_hbm, srcp_hbm, dstp_hbm, zrows_hbm, zdeg_hbm,
              out_acc_hbm, out_deg_hbm, src_v, dst_v, buf_v, ones_v, acc_sh,
              deg_sh, sem):
    c = lax.axis_index("c")
    s = lax.axis_index("s")
    w = c * NS + s
    row0 = s * RPT
    pltpu.sync_copy(zrows_hbm.at[pl.ds(row0, RPT)], acc_sh.at[pl.ds(row0, RPT)])
    if count_deg:
        pltpu.sync_copy(zdeg_hbm.at[pl.ds(row0, RPT)], deg_sh.at[pl.ds(row0, RPT)])

        def fill_ones(i, carry):
            ones_v[pl.ds(i * LANES, LANES)] = jnp.ones((LANES,), _f32)
            return carry

        lax.fori_loop(0, B // LANES, fill_ones, 0)
    pltpu.sync_copy(srcp_hbm.at[w], src_v)
    pltpu.sync_copy(dstp_hbm.at[w], dst_v)
    plsc.subcore_barrier()

    def chunk(j, carry):
        pltpu.async_copy(x_hbm.at[src_v.at[j]], buf_v, sem).wait()
        pltpu.sync_copy(buf_v, acc_sh.at[dst_v.at[j]], add=True)
        if count_deg:
            pltpu.sync_copy(ones_v, deg_sh.at[dst_v.at[j]], add=True)
        return carry

    lax.fori_loop(0, NCHUNK, chunk, 0)
    plsc.subcore_barrier()
    pltpu.sync_copy(acc_sh.at[pl.ds(row0, RPT)],
                    out_acc_hbm.at[c, pl.ds(row0, RPT)])
    if count_deg:
        pltpu.sync_copy(deg_sh.at[pl.ds(row0, RPT)],
                        out_deg_hbm.at[c, pl.ds(row0, RPT)])


def _make_agg(nfeat, count_deg):
    mesh = plsc.VectorSubcoreMesh(core_axis_name="c", subcore_axis_name="s")
    out_type = [jax.ShapeDtypeStruct((NC, R, nfeat), _f32)]
    if count_deg:
        out_type.append(jax.ShapeDtypeStruct((NC, R), _f32))
    scratch = [
        pltpu.VMEM((NCHUNK, B), jnp.int32),
        pltpu.VMEM((NCHUNK, B), jnp.int32),
        pltpu.VMEM((B, nfeat), _f32),
        pltpu.VMEM((B,), _f32),
        pltpu.VMEM_SHARED((R, nfeat), _f32),
        pltpu.VMEM_SHARED((R,), _f32),
        pltpu.SemaphoreType.DMA,
    ]

    if count_deg:
        def body(x_hbm, srcp, dstp, zrows, zdeg, out_acc, out_deg,
                 src_v, dst_v, buf_v, ones_v, acc_sh, deg_sh, sem):
            _agg_body(nfeat, True, x_hbm, srcp, dstp, zrows, zdeg,
                      out_acc, out_deg, src_v, dst_v, buf_v, ones_v,
                      acc_sh, deg_sh, sem)
    else:
        def body(x_hbm, srcp, dstp, zrows, out_acc,
                 src_v, dst_v, buf_v, ones_v, acc_sh, deg_sh, sem):
            _agg_body(nfeat, False, x_hbm, srcp, dstp, zrows, None,
                      out_acc, None, src_v, dst_v, buf_v, ones_v,
                      acc_sh, deg_sh, sem)

    params = None
    if nfeat % 128 != 0:
        params = pltpu.CompilerParams(use_tc_tiling_on_sc=False)
    return pl.kernel(body, out_type=tuple(out_type), mesh=mesh,
                     scratch_types=scratch, compiler_params=params)


_agg1 = _make_agg(IN_FEATS, True)
_agg2 = _make_agg(N_CLASSES, False)

R_B = 1280
_GRID = R // R_B


def _dense1_body(p_acc, p_deg, W1r, b1r, W2r, g_ref):
    a = p_acc[0] + p_acc[1]
    deg = jnp.maximum(p_deg[0] + p_deg[1], 1.0)
    a = a * (1.0 / deg)
    h = jnp.dot(a, W1r[...], preferred_element_type=_f32) + b1r[...]
    h = jnp.maximum(h, 0.0)
    g_ref[...] = jnp.dot(h, W2r[...], preferred_element_type=_f32)


def _dense2_body(q_ref, p_deg, b2r, o_ref):
    qsum = q_ref[0] + q_ref[1]
    deg = jnp.maximum(p_deg[0] + p_deg[1], 1.0)
    o_ref[...] = qsum / deg + b2r[...]


_dense1 = pl.pallas_call(
    _dense1_body,
    grid=(_GRID,),
    in_specs=[
        pl.BlockSpec((2, R_B, IN_FEATS), lambda i: (0, i, 0)),
        pl.BlockSpec((2, R_B, 1), lambda i: (0, i, 0)),
        pl.BlockSpec((IN_FEATS, N_HIDDEN), lambda i: (0, 0)),
        pl.BlockSpec((1, N_HIDDEN), lambda i: (0, 0)),
        pl.BlockSpec((N_HIDDEN, N_CLASSES), lambda i: (0, 0)),
    ],
    out_specs=pl.BlockSpec((R_B, N_CLASSES), lambda i: (i, 0)),
    out_shape=jax.ShapeDtypeStruct((R, N_CLASSES), _f32),
)

_dense2 = pl.pallas_call(
    _dense2_body,
    grid=(_GRID,),
    in_specs=[
        pl.BlockSpec((2, R_B, N_CLASSES), lambda i: (0, i, 0)),
        pl.BlockSpec((2, R_B, 1), lambda i: (0, i, 0)),
        pl.BlockSpec((1, N_CLASSES), lambda i: (0, 0)),
    ],
    out_specs=pl.BlockSpec((R_B, N_CLASSES), lambda i: (i, 0)),
    out_shape=jax.ShapeDtypeStruct((R, N_CLASSES), _f32),
)


def kernel(x, edge_index, W1, b1, W2, b2):
    src = edge_index[0].astype(jnp.int32)
    dst = edge_index[1].astype(jnp.int32)
    pad_n = E_PAD - N_EDGES
    pad_i = jnp.arange(pad_n, dtype=jnp.int32)
    src_p = jnp.concatenate([src, pad_i % N_NODES]).reshape(NW, NCHUNK, B)
    dst_p = jnp.concatenate([dst, N_NODES + pad_i % (R - N_NODES)]).reshape(
        NW, NCHUNK, B)
    zrows1 = jnp.zeros((R, IN_FEATS), _f32)
    zrows2 = jnp.zeros((R, N_CLASSES), _f32)
    zdeg = jnp.zeros((R,), _f32)

    p_acc, p_deg = _agg1(x, src_p, dst_p, zrows1, zdeg)
    p_deg3 = p_deg.reshape(NC, R, 1)
    g = _dense1(p_acc, p_deg3, W1, b1.reshape(1, N_HIDDEN), W2)
    (q,) = _agg2(g, src_p, dst_p, zrows2)
    out = _dense2(q, p_deg3, b2.reshape(1, N_CLASSES))
    return out[:N_NODES]


# speedup vs baseline: 13.0648x; 1.4488x over previous
import functools

import jax
import jax.numpy as jnp
from jax import lax
from jax.experimental import pallas as pl
from jax.experimental.pallas import tpu as pltpu
from jax.experimental.pallas import tpu_sc as plsc

N_NODES = 10000
IN_FEATS = 128
N_HIDDEN = 128
N_CLASSES = 64
N_EDGES = 320000

NC = 2
NS = 16
NW = NC * NS
LANES = 16

B = 128
NCHUNK = -(-N_EDGES // (NW * B))
NCHUNK = -(-NCHUNK // 4) * 4
E_PAD = NW * B * NCHUNK
RPT = 640
R = NS * RPT

_f32 = jnp.float32


def _agg_body(nfeat, count_deg, x_hbm, idxp_hbm, zrows_hbm, zdeg_hbm,
              out_acc_hbm, out_deg_hbm, idx_v, buf0_v, buf1_v, ones_v,
              acc_sh, deg_sh, gsem0, gsem1, isem0, isem1, isem2, isem3):
    c = lax.axis_index("c")
    s = lax.axis_index("s")
    w = c * NS + s
    row0 = s * RPT
    bufs = (buf0_v, buf1_v)
    gsems = (gsem0, gsem1)
    isems = (isem0, isem1, isem2, isem3)

    pltpu.sync_copy(zrows_hbm.at[pl.ds(row0, RPT)], acc_sh.at[pl.ds(row0, RPT)])
    if count_deg:
        pltpu.sync_copy(zdeg_hbm.at[pl.ds(row0, RPT)], deg_sh.at[pl.ds(row0, RPT)])

        def fill_ones(i, carry):
            ones_v[pl.ds(i * LANES, LANES)] = jnp.ones((LANES,), _f32)
            return carry

        lax.fori_loop(0, B // LANES, fill_ones, 0)
    plsc.subcore_barrier()

    def idx_start(j, k):
        pltpu.async_copy(idxp_hbm.at[w, j], idx_v.at[pl.ds(2 * k, 2)], isems[k])

    def idx_wait(j, k):
        pltpu.make_async_copy(idxp_hbm.at[w, j], idx_v.at[pl.ds(2 * k, 2)],
                              isems[k]).wait()

    def g_start(k, p):
        pltpu.async_copy(x_hbm.at[idx_v.at[2 * k]], bufs[p], gsems[p])

    def g_wait(k, p):
        pltpu.make_async_copy(x_hbm.at[idx_v.at[2 * k]], bufs[p],
                              gsems[p]).wait()

    def scat(k, p):
        pltpu.sync_copy(bufs[p], acc_sh.at[idx_v.at[2 * k + 1]], add=True)
        if count_deg:
            pltpu.sync_copy(ones_v, deg_sh.at[idx_v.at[2 * k + 1]], add=True)

    def step(j, k, last):
        kn = (k + 1) % 4
        p = k % 2
        if not last:
            idx_wait(j + 1, kn)
            g_start(kn, 1 - p)
        g_wait(k, p)
        scat(k, p)

    for k in range(4):
        idx_start(k, k)
    idx_wait(0, 0)
    g_start(0, 0)

    def quad(t, carry):
        a = 4 * t
        for k in range(4):
            step(a + k, k, False)
            idx_start(a + k + 4, k)
        return carry

    lax.fori_loop(0, NCHUNK // 4 - 1, quad, 0)
    a = NCHUNK - 4
    for k in range(4):
        step(a + k, k, last=(k == 3))

    plsc.subcore_barrier()
    pltpu.sync_copy(acc_sh.at[pl.ds(row0, RPT)],
                    out_acc_hbm.at[c, pl.ds(row0, RPT)])
    if count_deg:
        pltpu.sync_copy(deg_sh.at[pl.ds(row0, RPT)],
                        out_deg_hbm.at[c, pl.ds(row0, RPT)])


def _make_agg(nfeat, count_deg):
    mesh = plsc.VectorSubcoreMesh(core_axis_name="c", subcore_axis_name="s")
    out_type = [jax.ShapeDtypeStruct((NC, R, nfeat), _f32)]
    if count_deg:
        out_type.append(jax.ShapeDtypeStruct((NC, R), _f32))
    scratch = [
        pltpu.VMEM((8, B), jnp.int32),
        pltpu.VMEM((B, nfeat), _f32),
        pltpu.VMEM((B, nfeat), _f32),
        pltpu.VMEM((B,), _f32),
        pltpu.VMEM_SHARED((R, nfeat), _f32),
        pltpu.VMEM_SHARED((R,), _f32),
        pltpu.SemaphoreType.DMA,
        pltpu.SemaphoreType.DMA,
        pltpu.SemaphoreType.DMA,
        pltpu.SemaphoreType.DMA,
        pltpu.SemaphoreType.DMA,
        pltpu.SemaphoreType.DMA,
    ]

    if count_deg:
        def body(x_hbm, idxp, zrows, zdeg, out_acc, out_deg,
                 idx_v, buf0_v, buf1_v, ones_v, acc_sh, deg_sh,
                 gsem0, gsem1, isem0, isem1, isem2, isem3):
            _agg_body(nfeat, True, x_hbm, idxp, zrows, zdeg,
                      out_acc, out_deg, idx_v, buf0_v, buf1_v,
                      ones_v, acc_sh, deg_sh, gsem0, gsem1,
                      isem0, isem1, isem2, isem3)
    else:
        def body(x_hbm, idxp, zrows, out_acc,
                 idx_v, buf0_v, buf1_v, ones_v, acc_sh, deg_sh,
                 gsem0, gsem1, isem0, isem1, isem2, isem3):
            _agg_body(nfeat, False, x_hbm, idxp, zrows, None,
                      out_acc, None, idx_v, buf0_v, buf1_v,
                      ones_v, acc_sh, deg_sh, gsem0, gsem1,
                      isem0, isem1, isem2, isem3)

    params = None
    if nfeat % 128 != 0:
        params = pltpu.CompilerParams(use_tc_tiling_on_sc=False)
    return pl.kernel(body, out_type=tuple(out_type), mesh=mesh,
                     scratch_types=scratch, compiler_params=params)


_agg1 = _make_agg(IN_FEATS, True)
_agg2 = _make_agg(N_CLASSES, False)

R_B = 1280
_GRID = R // R_B


def _dense1_body(p_acc, p_deg, W1r, b1r, W2r, g_ref):
    a = p_acc[0] + p_acc[1]
    deg = jnp.maximum(p_deg[0] + p_deg[1], 1.0)
    a = a * (1.0 / deg)
    h = jnp.dot(a, W1r[...], preferred_element_type=_f32) + b1r[...]
    h = jnp.maximum(h, 0.0)
    g_ref[...] = jnp.dot(h, W2r[...], preferred_element_type=_f32)


def _dense2_body(q_ref, p_deg, b2r, o_ref):
    qsum = q_ref[0] + q_ref[1]
    deg = jnp.maximum(p_deg[0] + p_deg[1], 1.0)
    o_ref[...] = qsum / deg + b2r[...]


_dense1 = pl.pallas_call(
    _dense1_body,
    grid=(_GRID,),
    in_specs=[
        pl.BlockSpec((2, R_B, IN_FEATS), lambda i: (0, i, 0)),
        pl.BlockSpec((2, R_B, 1), lambda i: (0, i, 0)),
        pl.BlockSpec((IN_FEATS, N_HIDDEN), lambda i: (0, 0)),
        pl.BlockSpec((1, N_HIDDEN), lambda i: (0, 0)),
        pl.BlockSpec((N_HIDDEN, N_CLASSES), lambda i: (0, 0)),
    ],
    out_specs=pl.BlockSpec((R_B, N_CLASSES), lambda i: (i, 0)),
    out_shape=jax.ShapeDtypeStruct((R, N_CLASSES), _f32),
)

_dense2 = pl.pallas_call(
    _dense2_body,
    grid=(_GRID,),
    in_specs=[
        pl.BlockSpec((2, R_B, N_CLASSES), lambda i: (0, i, 0)),
        pl.BlockSpec((2, R_B, 1), lambda i: (0, i, 0)),
        pl.BlockSpec((1, N_CLASSES), lambda i: (0, 0)),
    ],
    out_specs=pl.BlockSpec((R_B, N_CLASSES), lambda i: (i, 0)),
    out_shape=jax.ShapeDtypeStruct((R, N_CLASSES), _f32),
)


def kernel(x, edge_index, W1, b1, W2, b2):
    src = edge_index[0].astype(jnp.int32)
    dst = edge_index[1].astype(jnp.int32)
    pad_n = E_PAD - N_EDGES
    pad_i = jnp.arange(pad_n, dtype=jnp.int32)
    src_p = jnp.concatenate([src, pad_i % N_NODES]).reshape(NW, NCHUNK, B)
    dst_p = jnp.concatenate([dst, N_NODES + pad_i % (R - N_NODES)]).reshape(
        NW, NCHUNK, B)
    idxp = jnp.stack([src_p, dst_p], axis=2)
    zrows1 = jnp.zeros((R, IN_FEATS), _f32)
    zrows2 = jnp.zeros((R, N_CLASSES), _f32)
    zdeg = jnp.zeros((R,), _f32)

    p_acc, p_deg = _agg1(x, idxp, zrows1, zdeg)
    p_deg3 = p_deg.reshape(NC, R, 1)
    g = _dense1(p_acc, p_deg3, W1, b1.reshape(1, N_HIDDEN), W2)
    (q,) = _agg2(g, idxp, zrows2)
    out = _dense2(q, p_deg3, b2.reshape(1, N_CLASSES))
    return out[:N_NODES]


# speedup vs baseline: 13.1084x; 1.0033x over previous
import functools

import jax
import jax.numpy as jnp
from jax import lax
from jax.experimental import pallas as pl
from jax.experimental.pallas import tpu as pltpu
from jax.experimental.pallas import tpu_sc as plsc

N_NODES = 10000
IN_FEATS = 128
N_HIDDEN = 128
N_CLASSES = 64
N_EDGES = 320000

NC = 2
NS = 16
NW = NC * NS
LANES = 16

B = 128
NCHUNK = -(-N_EDGES // (NW * B))
NCHUNK = -(-NCHUNK // 4) * 4
E_PAD = NW * B * NCHUNK
RPT = 640
R = NS * RPT

_f32 = jnp.float32


def _agg_body(nfeat, count_deg, x_hbm, idxp_hbm, zrows_hbm, zdeg_hbm,
              out_acc_hbm, out_deg_hbm, idx_v, buf0_v, buf1_v, ones_v,
              acc_sh, deg_sh, gsem0, gsem1, isem0, isem1, isem2, isem3,
              ssem0, ssem1, osem0, osem1):
    c = lax.axis_index("c")
    s = lax.axis_index("s")
    w = c * NS + s
    row0 = s * RPT
    bufs = (buf0_v, buf1_v)
    gsems = (gsem0, gsem1)
    isems = (isem0, isem1, isem2, isem3)
    ssems = (ssem0, ssem1)
    osems = (osem0, osem1)

    pltpu.sync_copy(zrows_hbm.at[pl.ds(row0, RPT)], acc_sh.at[pl.ds(row0, RPT)])
    if count_deg:
        pltpu.sync_copy(zdeg_hbm.at[pl.ds(row0, RPT)], deg_sh.at[pl.ds(row0, RPT)])

        def fill_ones(i, carry):
            ones_v[pl.ds(i * LANES, LANES)] = jnp.ones((LANES,), _f32)
            return carry

        lax.fori_loop(0, B // LANES, fill_ones, 0)
    plsc.subcore_barrier()

    def idx_start(j, k):
        pltpu.async_copy(idxp_hbm.at[w, j], idx_v.at[pl.ds(2 * k, 2)], isems[k])

    def idx_wait(j, k):
        pltpu.make_async_copy(idxp_hbm.at[w, j], idx_v.at[pl.ds(2 * k, 2)],
                              isems[k]).wait()

    def g_start(k, p):
        pltpu.async_copy(x_hbm.at[idx_v.at[2 * k]], bufs[p], gsems[p])

    def g_wait(k, p):
        pltpu.make_async_copy(x_hbm.at[idx_v.at[2 * k]], bufs[p],
                              gsems[p]).wait()

    def scat(k, p):
        pltpu.async_copy(bufs[p], acc_sh.at[idx_v.at[2 * k + 1]], ssems[p],
                         add=True)
        if count_deg:
            pltpu.async_copy(ones_v, deg_sh.at[idx_v.at[2 * k + 1]], osems[p],
                             add=True)

    def scat_wait(km, p):
        pltpu.make_async_copy(bufs[p], acc_sh.at[idx_v.at[2 * km + 1]],
                              ssems[p]).wait()
        if count_deg:
            pltpu.make_async_copy(ones_v, deg_sh.at[idx_v.at[2 * km + 1]],
                                  osems[p]).wait()

    def step(j, k, first=False, last=False, refill=True):
        kn = (k + 1) % 4
        km = (k + 3) % 4
        p = k % 2
        if not last:
            idx_wait(j + 1, kn)
        if not first:
            scat_wait(km, 1 - p)
        if not last:
            g_start(kn, 1 - p)
        if not first and refill:
            idx_start(j + 3, km)
        g_wait(k, p)
        scat(k, p)

    for k in range(4):
        idx_start(k, k)
    idx_wait(0, 0)
    g_start(0, 0)

    for k in range(4):
        step(k, k, first=(k == 0))

    def quad(t, carry):
        a = 4 * t
        for k in range(4):
            step(a + k, k)
        return carry

    lax.fori_loop(1, NCHUNK // 4 - 1, quad, 0)
    a = NCHUNK - 4
    for k in range(4):
        step(a + k, k, last=(k == 3), refill=(k == 0))
    scat_wait(3, (NCHUNK - 1) % 2)

    plsc.subcore_barrier()
    pltpu.sync_copy(acc_sh.at[pl.ds(row0, RPT)],
                    out_acc_hbm.at[c, pl.ds(row0, RPT)])
    if count_deg:
        pltpu.sync_copy(deg_sh.at[pl.ds(row0, RPT)],
                        out_deg_hbm.at[c, pl.ds(row0, RPT)])


def _make_agg(nfeat, count_deg):
    mesh = plsc.VectorSubcoreMesh(core_axis_name="c", subcore_axis_name="s")
    out_type = [jax.ShapeDtypeStruct((NC, R, nfeat), _f32)]
    if count_deg:
        out_type.append(jax.ShapeDtypeStruct((NC, R), _f32))
    scratch = [
        pltpu.VMEM((8, B), jnp.int32),
        pltpu.VMEM((B, nfeat), _f32),
        pltpu.VMEM((B, nfeat), _f32),
        pltpu.VMEM((B,), _f32),
        pltpu.VMEM_SHARED((R, nfeat), _f32),
        pltpu.VMEM_SHARED((R,), _f32),
        pltpu.SemaphoreType.DMA,
        pltpu.SemaphoreType.DMA,
        pltpu.SemaphoreType.DMA,
        pltpu.SemaphoreType.DMA,
        pltpu.SemaphoreType.DMA,
        pltpu.SemaphoreType.DMA,
        pltpu.SemaphoreType.DMA,
        pltpu.SemaphoreType.DMA,
        pltpu.SemaphoreType.DMA,
        pltpu.SemaphoreType.DMA,
    ]

    if count_deg:
        def body(x_hbm, idxp, zrows, zdeg, out_acc, out_deg,
                 idx_v, buf0_v, buf1_v, ones_v, acc_sh, deg_sh,
                 gsem0, gsem1, isem0, isem1, isem2, isem3,
                 ssem0, ssem1, osem0, osem1):
            _agg_body(nfeat, True, x_hbm, idxp, zrows, zdeg,
                      out_acc, out_deg, idx_v, buf0_v, buf1_v,
                      ones_v, acc_sh, deg_sh, gsem0, gsem1,
                      isem0, isem1, isem2, isem3,
                      ssem0, ssem1, osem0, osem1)
    else:
        def body(x_hbm, idxp, zrows, out_acc,
                 idx_v, buf0_v, buf1_v, ones_v, acc_sh, deg_sh,
                 gsem0, gsem1, isem0, isem1, isem2, isem3,
                 ssem0, ssem1, osem0, osem1):
            _agg_body(nfeat, False, x_hbm, idxp, zrows, None,
                      out_acc, None, idx_v, buf0_v, buf1_v,
                      ones_v, acc_sh, deg_sh, gsem0, gsem1,
                      isem0, isem1, isem2, isem3,
                      ssem0, ssem1, osem0, osem1)

    params = None
    if nfeat % 128 != 0:
        params = pltpu.CompilerParams(use_tc_tiling_on_sc=False)
    return pl.kernel(body, out_type=tuple(out_type), mesh=mesh,
                     scratch_types=scratch, compiler_params=params)


_agg1 = _make_agg(IN_FEATS, True)
_agg2 = _make_agg(N_CLASSES, False)

R_B = 1280
_GRID = R // R_B


def _dense1_body(p_acc, p_deg, W1r, b1r, W2r, g_ref):
    a = p_acc[0] + p_acc[1]
    deg = jnp.maximum(p_deg[0] + p_deg[1], 1.0)
    a = a * (1.0 / deg)
    h = jnp.dot(a, W1r[...], preferred_element_type=_f32) + b1r[...]
    h = jnp.maximum(h, 0.0)
    g_ref[...] = jnp.dot(h, W2r[...], preferred_element_type=_f32)


def _dense2_body(q_ref, p_deg, b2r, o_ref):
    qsum = q_ref[0] + q_ref[1]
    deg = jnp.maximum(p_deg[0] + p_deg[1], 1.0)
    o_ref[...] = qsum / deg + b2r[...]


_dense1 = pl.pallas_call(
    _dense1_body,
    grid=(_GRID,),
    in_specs=[
        pl.BlockSpec((2, R_B, IN_FEATS), lambda i: (0, i, 0)),
        pl.BlockSpec((2, R_B, 1), lambda i: (0, i, 0)),
        pl.BlockSpec((IN_FEATS, N_HIDDEN), lambda i: (0, 0)),
        pl.BlockSpec((1, N_HIDDEN), lambda i: (0, 0)),
        pl.BlockSpec((N_HIDDEN, N_CLASSES), lambda i: (0, 0)),
    ],
    out_specs=pl.BlockSpec((R_B, N_CLASSES), lambda i: (i, 0)),
    out_shape=jax.ShapeDtypeStruct((R, N_CLASSES), _f32),
)

_dense2 = pl.pallas_call(
    _dense2_body,
    grid=(_GRID,),
    in_specs=[
        pl.BlockSpec((2, R_B, N_CLASSES), lambda i: (0, i, 0)),
        pl.BlockSpec((2, R_B, 1), lambda i: (0, i, 0)),
        pl.BlockSpec((1, N_CLASSES), lambda i: (0, 0)),
    ],
    out_specs=pl.BlockSpec((R_B, N_CLASSES), lambda i: (i, 0)),
    out_shape=jax.ShapeDtypeStruct((R, N_CLASSES), _f32),
)


def kernel(x, edge_index, W1, b1, W2, b2):
    src = edge_index[0].astype(jnp.int32)
    dst = edge_index[1].astype(jnp.int32)
    pad_n = E_PAD - N_EDGES
    pad_i = jnp.arange(pad_n, dtype=jnp.int32)
    src_p = jnp.concatenate([src, pad_i % N_NODES]).reshape(NW, NCHUNK, B)
    dst_p = jnp.concatenate([dst, N_NODES + pad_i % (R - N_NODES)]).reshape(
        NW, NCHUNK, B)
    idxp = jnp.stack([src_p, dst_p], axis=2)
    zrows1 = jnp.zeros((R, IN_FEATS), _f32)
    zrows2 = jnp.zeros((R, N_CLASSES), _f32)
    zdeg = jnp.zeros((R,), _f32)

    p_acc, p_deg = _agg1(x, idxp, zrows1, zdeg)
    p_deg3 = p_deg.reshape(NC, R, 1)
    g = _dense1(p_acc, p_deg3, W1, b1.reshape(1, N_HIDDEN), W2)
    (q,) = _agg2(g, idxp, zrows2)
    out = _dense2(q, p_deg3, b2.reshape(1, N_CLASSES))
    return out[:N_NODES]
